# baseline (speedup 1.0000x reference)
import functools

import jax
import jax.numpy as jnp
from jax import lax
from jax.experimental import pallas as pl
from jax.experimental.pallas import tpu as pltpu
from jax.experimental.pallas import tpu_sc as plsc

N = 10000
DEG = 32
DN = 128
DE = 16
E = N * DEG
DOUT = 128
NMOL = 256

NC, NS, L = 2, 16, 16
NW = NC * NS
PER_W = 320
A = 8
CHUNKS = PER_W // A
ROWS = A * DEG
NCOL = DN // L
LAST_W = NW - 1
LAST_N = N - LAST_W * PER_W
LAST_CHUNKS = LAST_N // A

BLK = 2000
NB = N // BLK


def _sc_gather_sums_body(atbl, btbl, aidx, bidx, asum, bsum,
                         idxt_a, idxt_b, idx_a, idx_b, arows, brows,
                         aout, bout, sga, sgb, swr):
    wid = lax.axis_index("s") * NC + lax.axis_index("c")
    base = wid * PER_W
    nch = jnp.where(wid == LAST_W, LAST_CHUNKS, CHUNKS)

    @pl.when(wid != LAST_W)
    def _():
        pltpu.sync_copy(aidx.at[pl.ds(base, PER_W)], idxt_a)
        pltpu.sync_copy(bidx.at[pl.ds(base, PER_W)], idxt_b)

    @pl.when(wid == LAST_W)
    def _():
        pltpu.sync_copy(aidx.at[pl.ds(LAST_W * PER_W, LAST_N)],
                        idxt_a.at[pl.ds(0, LAST_N)])
        pltpu.sync_copy(bidx.at[pl.ds(LAST_W * PER_W, LAST_N)],
                        idxt_b.at[pl.ds(0, LAST_N)])

    def reformat(i, carry):
        for h in range(DEG // L):
            sl = pl.ds(h * L, L)
            idx_a[pl.ds(i * DEG + h * L, L)] = idxt_a[i, sl]
            idx_b[pl.ds(i * DEG + h * L, L)] = idxt_b[i, sl]
        return carry

    lax.fori_loop(0, PER_W, reformat, 0)

    def fire(g, buf):
        for h in range(2):
            sl = pl.ds(g * ROWS + h * 128, 128)
            dl = pl.ds(h * 128, 128)
            pltpu.async_copy(atbl.at[idx_a.at[sl]], arows.at[buf].at[dl],
                             sga.at[buf])
            pltpu.async_copy(btbl.at[idx_b.at[sl]], brows.at[buf].at[dl],
                             sgb.at[buf])

    def drain_gathers(g, buf):
        for h in range(2):
            sl = pl.ds(g * ROWS + h * 128, 128)
            dl = pl.ds(h * 128, 128)
            pltpu.make_async_copy(atbl.at[idx_a.at[sl]],
                                  arows.at[buf].at[dl], sga.at[buf]).wait()
            pltpu.make_async_copy(btbl.at[idx_b.at[sl]],
                                  brows.at[buf].at[dl], sgb.at[buf]).wait()

    def drain_write(g, buf):
        abase = base + g * A
        pltpu.make_async_copy(aout.at[buf], asum.at[pl.ds(abase, A)],
                              swr.at[buf]).wait()
        pltpu.make_async_copy(bout.at[buf], bsum.at[pl.ds(abase, A)],
                              swr.at[buf]).wait()

    fire(0, 0)

    def chunk_body(g, carry):
        buf = lax.rem(g, 2)

        @pl.when(g + 1 < nch)
        def _():
            fire(g + 1, lax.rem(g + 1, 2))

        drain_gathers(g, buf)

        @pl.when(g >= 2)
        def _():
            drain_write(g - 2, buf)

        for a in range(A):
            rb = a * DEG
            for c in range(NCOL):
                dl = pl.ds(c * L, L)
                accs = [arows[buf, rb + k, dl] for k in range(4)]
                for r in range(4, DEG):
                    accs[r % 4] += arows[buf, rb + r, dl]
                aout[buf, a, dl] = (accs[0] + accs[1]) + (accs[2] + accs[3])
            baccs = [brows[buf, rb + k] for k in range(4)]
            for r in range(4, DEG):
                baccs[r % 4] += brows[buf, rb + r]
            bout[buf, a] = (baccs[0] + baccs[1]) + (baccs[2] + baccs[3])

        abase = base + g * A
        pltpu.async_copy(aout.at[buf], asum.at[pl.ds(abase, A)], swr.at[buf])
        pltpu.async_copy(bout.at[buf], bsum.at[pl.ds(abase, A)], swr.at[buf])
        return carry

    lax.fori_loop(0, nch, chunk_body, 0)
    drain_write(nch - 2, lax.rem(nch - 2, 2))
    drain_write(nch - 1, lax.rem(nch - 1, 2))


@functools.cache
def _get_sc_kernel():
    mesh = plsc.VectorSubcoreMesh(
        core_axis_name="c", subcore_axis_name="s",
        num_cores=NC, num_subcores=NS)
    return pl.kernel(
        _sc_gather_sums_body,
        out_type=(
            jax.ShapeDtypeStruct((N, DN), jnp.float32),
            jax.ShapeDtypeStruct((N, DE), jnp.float32),
        ),
        mesh=mesh,
        scratch_types=[
            pltpu.VMEM((PER_W, DEG), jnp.int32),
            pltpu.VMEM((PER_W, DEG), jnp.int32),
            pltpu.VMEM((PER_W * DEG,), jnp.int32),
            pltpu.VMEM((PER_W * DEG,), jnp.int32),
            pltpu.VMEM((2, ROWS, DN), jnp.float32),
            pltpu.VMEM((2, ROWS, DE), jnp.float32),
            pltpu.VMEM((2, A, DN), jnp.float32),
            pltpu.VMEM((2, A, DE), jnp.float32),
            pltpu.SemaphoreType.DMA((2,)),
            pltpu.SemaphoreType.DMA((2,)),
            pltpu.SemaphoreType.DMA((2,)),
        ],
        compiler_params=pltpu.CompilerParams(use_tc_tiling_on_sc=False),
    )


def _dot_t(x, w):
    return lax.dot_general(x, w, (((1,), (1,)), ((), ())),
                           preferred_element_type=jnp.float32)


def _onehot_t(mol_row):
    seg = lax.broadcasted_iota(jnp.int32, (NMOL, BLK), 0)
    return jnp.where(mol_row == seg, 1.0, 0.0)


def _tc1_body(ar_ref, asum_ref, bsum_ref, mol_ref, wdeg_ref, wself_ref,
              bias_ref, wout0_ref, bout0_ref, act_ref, stats_ref, fp0_ref):
    b = pl.program_id(0)
    ar = ar_ref[...]
    wdeg = wdeg_ref[...]
    wa = wdeg[:, :DN]
    wb = wdeg[:, DN:]
    wc = wa + wself_ref[...]
    act = (_dot_t(asum_ref[...], wa) + _dot_t(bsum_ref[...], wb)
           + _dot_t(ar, wc) + bias_ref[...])
    act_ref[...] = act

    psum = jnp.sum(act, axis=0, keepdims=True)
    psq = jnp.sum(act * act, axis=0, keepdims=True)

    logits = _dot_t(ar, wout0_ref[...]) + bout0_ref[...]
    m = jnp.max(logits, axis=1, keepdims=True)
    ex = jnp.exp(logits - m)
    soft = ex / jnp.sum(ex, axis=1, keepdims=True)
    oht = _onehot_t(mol_ref[0])
    fp_part = lax.dot_general(oht, soft, (((1,), (0,)), ((), ())),
                              preferred_element_type=jnp.float32)

    @pl.when(b == 0)
    def _():
        stats_ref[...] = jnp.zeros((2, DN), jnp.float32)
        fp0_ref[...] = jnp.zeros((NMOL, DOUT), jnp.float32)

    stats_ref[0:1, :] += psum
    stats_ref[1:2, :] += psq
    fp0_ref[...] += fp_part


_tc1 = pl.pallas_call(
    _tc1_body,
    grid=(NB,),
    in_specs=[
        pl.BlockSpec((BLK, DN), lambda b: (b, 0)),
        pl.BlockSpec((BLK, DN), lambda b: (b, 0)),
        pl.BlockSpec((BLK, DE), lambda b: (b, 0)),
        pl.BlockSpec((1, 1, BLK), lambda b: (b, 0, 0)),
        pl.BlockSpec((DOUT, DN + DE), lambda b: (0, 0)),
        pl.BlockSpec((DOUT, DN), lambda b: (0, 0)),
        pl.BlockSpec((1, DOUT), lambda b: (0, 0)),
        pl.BlockSpec((DOUT, DN), lambda b: (0, 0)),
        pl.BlockSpec((1, DOUT), lambda b: (0, 0)),
    ],
    out_specs=[
        pl.BlockSpec((BLK, DN), lambda b: (b, 0)),
        pl.BlockSpec((2, DN), lambda b: (0, 0)),
        pl.BlockSpec((NMOL, DOUT), lambda b: (0, 0)),
    ],
    out_shape=[
        jax.ShapeDtypeStruct((N, DN), jnp.float32),
        jax.ShapeDtypeStruct((2, DN), jnp.float32),
        jax.ShapeDtypeStruct((NMOL, DOUT), jnp.float32),
    ],
)


def _tc2_body(act_ref, mol_ref, stats_ref, fp0_ref, wout1_ref, bout1_ref,
              out_ref):
    b = pl.program_id(0)
    mean = stats_ref[0:1, :] * (1.0 / N)
    var = stats_ref[1:2, :] * (1.0 / N) - mean * mean
    h = jnp.maximum((act_ref[...] - mean) * lax.rsqrt(var + 1e-5), 0.0)
    logits = _dot_t(h, wout1_ref[...]) + bout1_ref[...]
    m = jnp.max(logits, axis=1, keepdims=True)
    ex = jnp.exp(logits - m)
    soft = ex / jnp.sum(ex, axis=1, keepdims=True)
    oht = _onehot_t(mol_ref[0])
    fp_part = lax.dot_general(oht, soft, (((1,), (0,)), ((), ())),
                              preferred_element_type=jnp.float32)

    @pl.when(b == 0)
    def _():
        out_ref[...] = fp0_ref[...]

    out_ref[...] += fp_part


_tc2 = pl.pallas_call(
    _tc2_body,
    grid=(NB,),
    in_specs=[
        pl.BlockSpec((BLK, DN), lambda b: (b, 0)),
        pl.BlockSpec((1, 1, BLK), lambda b: (b, 0, 0)),
        pl.BlockSpec((2, DN), lambda b: (0, 0)),
        pl.BlockSpec((NMOL, DOUT), lambda b: (0, 0)),
        pl.BlockSpec((DOUT, DOUT), lambda b: (0, 0)),
        pl.BlockSpec((1, DOUT), lambda b: (0, 0)),
    ],
    out_specs=pl.BlockSpec((NMOL, DOUT), lambda b: (0, 0)),
    out_shape=jax.ShapeDtypeStruct((NMOL, DOUT), jnp.float32),
)


def kernel(atom_repr, bond_repr, atom_nbr_idx, bond_nbr_idx, mol_ids,
           W_deg, W_self, bias, W_out0, b_out0, W_out1, b_out1):
    asum, bsum = _get_sc_kernel()(atom_repr, bond_repr,
                                  atom_nbr_idx, bond_nbr_idx)

    mol3 = mol_ids.astype(jnp.int32).reshape(NB, 1, BLK)
    act, stats, fp0 = _tc1(atom_repr, asum, bsum, mol3, W_deg, W_self, bias,
                           W_out0, b_out0.reshape(1, DOUT))
    return _tc2(act, mol3, stats, fp0, W_out1, b_out1.reshape(1, DOUT))


# speedup vs baseline: 1.2266x; 1.2266x over previous
import functools

import jax
import jax.numpy as jnp
from jax import lax
from jax.experimental import pallas as pl
from jax.experimental.pallas import tpu as pltpu
from jax.experimental.pallas import tpu_sc as plsc

N = 10000
DEG = 32
DN = 128
DE = 16
E = N * DEG
DOUT = 128
NMOL = 256

NC, NS, L = 2, 16, 16
NW = NC * NS
PER_W = 320
A = 8
CHUNKS = PER_W // A
ROWS = A * DEG
NCOL = DN // L
LAST_W = NW - 1
LAST_N = N - LAST_W * PER_W
LAST_CHUNKS = LAST_N // A

BLK = 2000
NB = N // BLK


def _sc_gather_sums_body(atbl, btbl, aidx, bidx, asum, bsum,
                         idxt_a, idxt_b, idx_a, idx_b, arows, brows,
                         dest, za, zb, sacc, sbacc, sga, sgb, swr):
    sid = lax.axis_index("s")
    wid = sid * NC + lax.axis_index("c")
    base = wid * PER_W
    nch = jnp.where(wid == LAST_W, LAST_CHUNKS, CHUNKS)

    @pl.when(wid != LAST_W)
    def _():
        pltpu.sync_copy(aidx.at[pl.ds(base, PER_W)], idxt_a)
        pltpu.sync_copy(bidx.at[pl.ds(base, PER_W)], idxt_b)

    @pl.when(wid == LAST_W)
    def _():
        pltpu.sync_copy(aidx.at[pl.ds(LAST_W * PER_W, LAST_N)],
                        idxt_a.at[pl.ds(0, LAST_N)])
        pltpu.sync_copy(bidx.at[pl.ds(LAST_W * PER_W, LAST_N)],
                        idxt_b.at[pl.ds(0, LAST_N)])

    def reformat(i, carry):
        for h in range(DEG // L):
            sl = pl.ds(h * L, L)
            idx_a[pl.ds(i * DEG + h * L, L)] = idxt_a[i, sl]
            idx_b[pl.ds(i * DEG + h * L, L)] = idxt_b[i, sl]
        return carry

    lax.fori_loop(0, PER_W, reformat, 0)

    def fire(g, buf):
        for h in range(2):
            sl = pl.ds(g * ROWS + h * 128, 128)
            dl = pl.ds(h * 128, 128)
            pltpu.async_copy(atbl.at[idx_a.at[sl]], arows.at[buf].at[dl],
                             sga.at[buf])
            pltpu.async_copy(btbl.at[idx_b.at[sl]], brows.at[buf].at[dl],
                             sgb.at[buf])

    def drain_gathers(g, buf):
        for h in range(2):
            sl = pl.ds(g * ROWS + h * 128, 128)
            dl = pl.ds(h * 128, 128)
            pltpu.make_async_copy(atbl.at[idx_a.at[sl]],
                                  arows.at[buf].at[dl], sga.at[buf]).wait()
            pltpu.make_async_copy(btbl.at[idx_b.at[sl]],
                                  brows.at[buf].at[dl], sgb.at[buf]).wait()

    def srow(buf):
        return sid * (2 * A) + buf * A

    def drain_write(g, buf):
        abase = base + g * A
        rb = srow(buf)
        pltpu.make_async_copy(sacc.at[pl.ds(rb, A)],
                              asum.at[pl.ds(abase, A)], swr.at[buf]).wait()
        pltpu.make_async_copy(sbacc.at[pl.ds(rb, A)],
                              bsum.at[pl.ds(abase, A)], swr.at[buf]).wait()

    for i in range(A):
        for c in range(NCOL):
            za[i, pl.ds(c * L, L)] = jnp.zeros((L,), jnp.float32)
        zb[i] = jnp.zeros((L,), jnp.float32)
    for buf in range(2):
        for v in range(ROWS // L):
            dest[buf, pl.ds(v * L, L)] = jnp.full(
                (L,), srow(buf) + v // (DEG // L), jnp.int32)

    fire(0, 0)

    def chunk_body(g, carry):
        buf = lax.rem(g, 2)

        @pl.when(g + 1 < nch)
        def _():
            fire(g + 1, lax.rem(g + 1, 2))

        drain_gathers(g, buf)

        @pl.when(g >= 2)
        def _():
            drain_write(g - 2, buf)

        rb = srow(buf)
        pltpu.sync_copy(za, sacc.at[pl.ds(rb, A)])
        pltpu.sync_copy(zb, sbacc.at[pl.ds(rb, A)])
        pltpu.sync_copy(arows.at[buf], sacc.at[dest.at[buf]], add=True)
        pltpu.sync_copy(brows.at[buf], sbacc.at[dest.at[buf]], add=True)

        abase = base + g * A
        pltpu.async_copy(sacc.at[pl.ds(rb, A)], asum.at[pl.ds(abase, A)],
                         swr.at[buf])
        pltpu.async_copy(sbacc.at[pl.ds(rb, A)], bsum.at[pl.ds(abase, A)],
                         swr.at[buf])
        return carry

    lax.fori_loop(0, nch, chunk_body, 0)
    drain_write(nch - 2, lax.rem(nch - 2, 2))
    drain_write(nch - 1, lax.rem(nch - 1, 2))


@functools.cache
def _get_sc_kernel():
    mesh = plsc.VectorSubcoreMesh(
        core_axis_name="c", subcore_axis_name="s",
        num_cores=NC, num_subcores=NS)
    return pl.kernel(
        _sc_gather_sums_body,
        out_type=(
            jax.ShapeDtypeStruct((N, DN), jnp.float32),
            jax.ShapeDtypeStruct((N, DE), jnp.float32),
        ),
        mesh=mesh,
        scratch_types=[
            pltpu.VMEM((PER_W, DEG), jnp.int32),
            pltpu.VMEM((PER_W, DEG), jnp.int32),
            pltpu.VMEM((PER_W * DEG,), jnp.int32),
            pltpu.VMEM((PER_W * DEG,), jnp.int32),
            pltpu.VMEM((2, ROWS, DN), jnp.float32),
            pltpu.VMEM((2, ROWS, DE), jnp.float32),
            pltpu.VMEM((2, ROWS), jnp.int32),
            pltpu.VMEM((A, DN), jnp.float32),
            pltpu.VMEM((A, DE), jnp.float32),
            pltpu.VMEM_SHARED((NS * 2 * A, DN), jnp.float32),
            pltpu.VMEM_SHARED((NS * 2 * A, DE), jnp.float32),
            pltpu.SemaphoreType.DMA((2,)),
            pltpu.SemaphoreType.DMA((2,)),
            pltpu.SemaphoreType.DMA((2,)),
        ],
        compiler_params=pltpu.CompilerParams(use_tc_tiling_on_sc=False),
    )


def _dot_t(x, w):
    return lax.dot_general(x, w, (((1,), (1,)), ((), ())),
                           preferred_element_type=jnp.float32)


def _onehot_t(mol_row):
    seg = lax.broadcasted_iota(jnp.int32, (NMOL, BLK), 0)
    return jnp.where(mol_row == seg, 1.0, 0.0)


def _tc1_body(ar_ref, asum_ref, bsum_ref, mol_ref, wdeg_ref, wself_ref,
              bias_ref, wout0_ref, bout0_ref, act_ref, stats_ref, fp0_ref):
    b = pl.program_id(0)
    ar = ar_ref[...]
    wdeg = wdeg_ref[...]
    wa = wdeg[:, :DN]
    wb = wdeg[:, DN:]
    wc = wa + wself_ref[...]
    act = (_dot_t(asum_ref[...], wa) + _dot_t(bsum_ref[...], wb)
           + _dot_t(ar, wc) + bias_ref[...])
    act_ref[...] = act

    psum = jnp.sum(act, axis=0, keepdims=True)
    psq = jnp.sum(act * act, axis=0, keepdims=True)

    logits = _dot_t(ar, wout0_ref[...]) + bout0_ref[...]
    m = jnp.max(logits, axis=1, keepdims=True)
    ex = jnp.exp(logits - m)
    soft = ex / jnp.sum(ex, axis=1, keepdims=True)
    oht = _onehot_t(mol_ref[0])
    fp_part = lax.dot_general(oht, soft, (((1,), (0,)), ((), ())),
                              preferred_element_type=jnp.float32)

    @pl.when(b == 0)
    def _():
        stats_ref[...] = jnp.zeros((2, DN), jnp.float32)
        fp0_ref[...] = jnp.zeros((NMOL, DOUT), jnp.float32)

    stats_ref[0:1, :] += psum
    stats_ref[1:2, :] += psq
    fp0_ref[...] += fp_part


_tc1 = pl.pallas_call(
    _tc1_body,
    grid=(NB,),
    in_specs=[
        pl.BlockSpec((BLK, DN), lambda b: (b, 0)),
        pl.BlockSpec((BLK, DN), lambda b: (b, 0)),
        pl.BlockSpec((BLK, DE), lambda b: (b, 0)),
        pl.BlockSpec((1, 1, BLK), lambda b: (b, 0, 0)),
        pl.BlockSpec((DOUT, DN + DE), lambda b: (0, 0)),
        pl.BlockSpec((DOUT, DN), lambda b: (0, 0)),
        pl.BlockSpec((1, DOUT), lambda b: (0, 0)),
        pl.BlockSpec((DOUT, DN), lambda b: (0, 0)),
        pl.BlockSpec((1, DOUT), lambda b: (0, 0)),
    ],
    out_specs=[
        pl.BlockSpec((BLK, DN), lambda b: (b, 0)),
        pl.BlockSpec((2, DN), lambda b: (0, 0)),
        pl.BlockSpec((NMOL, DOUT), lambda b: (0, 0)),
    ],
    out_shape=[
        jax.ShapeDtypeStruct((N, DN), jnp.float32),
        jax.ShapeDtypeStruct((2, DN), jnp.float32),
        jax.ShapeDtypeStruct((NMOL, DOUT), jnp.float32),
    ],
)


def _tc2_body(act_ref, mol_ref, stats_ref, fp0_ref, wout1_ref, bout1_ref,
              out_ref):
    b = pl.program_id(0)
    mean = stats_ref[0:1, :] * (1.0 / N)
    var = stats_ref[1:2, :] * (1.0 / N) - mean * mean
    h = jnp.maximum((act_ref[...] - mean) * lax.rsqrt(var + 1e-5), 0.0)
    logits = _dot_t(h, wout1_ref[...]) + bout1_ref[...]
    m = jnp.max(logits, axis=1, keepdims=True)
    ex = jnp.exp(logits - m)
    soft = ex / jnp.sum(ex, axis=1, keepdims=True)
    oht = _onehot_t(mol_ref[0])
    fp_part = lax.dot_general(oht, soft, (((1,), (0,)), ((), ())),
                              preferred_element_type=jnp.float32)

    @pl.when(b == 0)
    def _():
        out_ref[...] = fp0_ref[...]

    out_ref[...] += fp_part


_tc2 = pl.pallas_call(
    _tc2_body,
    grid=(NB,),
    in_specs=[
        pl.BlockSpec((BLK, DN), lambda b: (b, 0)),
        pl.BlockSpec((1, 1, BLK), lambda b: (b, 0, 0)),
        pl.BlockSpec((2, DN), lambda b: (0, 0)),
        pl.BlockSpec((NMOL, DOUT), lambda b: (0, 0)),
        pl.BlockSpec((DOUT, DOUT), lambda b: (0, 0)),
        pl.BlockSpec((1, DOUT), lambda b: (0, 0)),
    ],
    out_specs=pl.BlockSpec((NMOL, DOUT), lambda b: (0, 0)),
    out_shape=jax.ShapeDtypeStruct((NMOL, DOUT), jnp.float32),
)


def kernel(atom_repr, bond_repr, atom_nbr_idx, bond_nbr_idx, mol_ids,
           W_deg, W_self, bias, W_out0, b_out0, W_out1, b_out1):
    asum, bsum = _get_sc_kernel()(atom_repr, bond_repr,
                                  atom_nbr_idx, bond_nbr_idx)

    mol3 = mol_ids.astype(jnp.int32).reshape(NB, 1, BLK)
    act, stats, fp0 = _tc1(atom_repr, asum, bsum, mol3, W_deg, W_self, bias,
                           W_out0, b_out0.reshape(1, DOUT))
    return _tc2(act, mol3, stats, fp0, W_out1, b_out1.reshape(1, DOUT))


# speedup vs baseline: 1.3128x; 1.0703x over previous
import functools

import jax
import jax.numpy as jnp
from jax import lax
from jax.experimental import pallas as pl
from jax.experimental.pallas import tpu as pltpu
from jax.experimental.pallas import tpu_sc as plsc

N = 10000
DEG = 32
DN = 128
DE = 16
E = N * DEG
DOUT = 128
NMOL = 256

NC, NS, L = 2, 16, 16
NW = NC * NS
PER_W = 320
A = 8
CHUNKS = PER_W // A
ROWS = A * DEG
NCOL = DN // L
LAST_W = NW - 1
LAST_N = N - LAST_W * PER_W
LAST_CHUNKS = LAST_N // A

BLK = 2000
NB = N // BLK


def _sc_gather_sums_body(atbl, btbl, aidx, bidx, asum, bsum,
                         idxt_a, idxt_b, idx_a, idx_b, arows, brows,
                         dest, za, zb, sacc, sbacc, sga, sgb, swr,
                         semz, sems):
    sid = lax.axis_index("s")
    wid = sid * NC + lax.axis_index("c")
    base = wid * PER_W
    nch = jnp.where(wid == LAST_W, LAST_CHUNKS, CHUNKS)

    @pl.when(wid != LAST_W)
    def _():
        pltpu.sync_copy(aidx.at[pl.ds(base, PER_W)], idxt_a)
        pltpu.sync_copy(bidx.at[pl.ds(base, PER_W)], idxt_b)

    @pl.when(wid == LAST_W)
    def _():
        pltpu.sync_copy(aidx.at[pl.ds(LAST_W * PER_W, LAST_N)],
                        idxt_a.at[pl.ds(0, LAST_N)])
        pltpu.sync_copy(bidx.at[pl.ds(LAST_W * PER_W, LAST_N)],
                        idxt_b.at[pl.ds(0, LAST_N)])

    def reformat(i, carry):
        for h in range(DEG // L):
            sl = pl.ds(h * L, L)
            idx_a[pl.ds(i * DEG + h * L, L)] = idxt_a[i, sl]
            idx_b[pl.ds(i * DEG + h * L, L)] = idxt_b[i, sl]
        return carry

    lax.fori_loop(0, PER_W, reformat, 0)

    def fire(g, buf):
        for h in range(2):
            sl = pl.ds(g * ROWS + h * 128, 128)
            dl = pl.ds(h * 128, 128)
            pltpu.async_copy(atbl.at[idx_a.at[sl]], arows.at[buf].at[dl],
                             sga.at[buf])
            pltpu.async_copy(btbl.at[idx_b.at[sl]], brows.at[buf].at[dl],
                             sgb.at[buf])

    def drain_gathers(g, buf):
        for h in range(2):
            sl = pl.ds(g * ROWS + h * 128, 128)
            dl = pl.ds(h * 128, 128)
            pltpu.make_async_copy(atbl.at[idx_a.at[sl]],
                                  arows.at[buf].at[dl], sga.at[buf]).wait()
            pltpu.make_async_copy(btbl.at[idx_b.at[sl]],
                                  brows.at[buf].at[dl], sgb.at[buf]).wait()

    def srow(buf):
        return sid * (2 * A) + buf * A

    def drain_write(g, buf):
        abase = base + g * A
        rb = srow(buf)
        pltpu.make_async_copy(sacc.at[pl.ds(rb, A)],
                              asum.at[pl.ds(abase, A)], swr.at[buf]).wait()
        pltpu.make_async_copy(sbacc.at[pl.ds(rb, A)],
                              bsum.at[pl.ds(abase, A)], swr.at[buf]).wait()

    for i in range(A):
        for c in range(NCOL):
            za[i, pl.ds(c * L, L)] = jnp.zeros((L,), jnp.float32)
        zb[i] = jnp.zeros((L,), jnp.float32)
    for buf in range(2):
        for v in range(ROWS // L):
            dest[buf, pl.ds(v * L, L)] = jnp.full(
                (L,), srow(buf) + v // (DEG // L), jnp.int32)

    def drain_scatter(buf):
        pltpu.make_async_copy(arows.at[buf], sacc.at[dest.at[buf]],
                              sems.at[buf]).wait()
        pltpu.make_async_copy(brows.at[buf], sbacc.at[dest.at[buf]],
                              sems.at[buf]).wait()

    def issue_write(g, buf):
        abase = base + g * A
        rb = srow(buf)
        pltpu.async_copy(sacc.at[pl.ds(rb, A)], asum.at[pl.ds(abase, A)],
                         swr.at[buf])
        pltpu.async_copy(sbacc.at[pl.ds(rb, A)], bsum.at[pl.ds(abase, A)],
                         swr.at[buf])

    fire(0, 0)

    def chunk_body(g, carry):
        buf = lax.rem(g, 2)
        nbuf = lax.rem(g + 1, 2)

        @pl.when(g + 1 < nch)
        def _():
            fire(g + 1, nbuf)

        @pl.when(g >= 1)
        def _():
            drain_scatter(nbuf)
            issue_write(g - 1, nbuf)

        @pl.when(g >= 2)
        def _():
            drain_write(g - 2, buf)

        rb = srow(buf)
        za_c = pltpu.async_copy(za, sacc.at[pl.ds(rb, A)], semz.at[buf])
        zb_c = pltpu.async_copy(zb, sbacc.at[pl.ds(rb, A)], semz.at[buf])
        drain_gathers(g, buf)
        za_c.wait()
        zb_c.wait()
        pltpu.async_copy(arows.at[buf], sacc.at[dest.at[buf]], sems.at[buf],
                         add=True)
        pltpu.async_copy(brows.at[buf], sbacc.at[dest.at[buf]], sems.at[buf],
                         add=True)
        return carry

    lax.fori_loop(0, nch, chunk_body, 0)
    lbuf = lax.rem(nch - 1, 2)
    drain_scatter(lbuf)
    issue_write(nch - 1, lbuf)
    drain_write(nch - 2, lax.rem(nch - 2, 2))
    drain_write(nch - 1, lbuf)


@functools.cache
def _get_sc_kernel():
    mesh = plsc.VectorSubcoreMesh(
        core_axis_name="c", subcore_axis_name="s",
        num_cores=NC, num_subcores=NS)
    return pl.kernel(
        _sc_gather_sums_body,
        out_type=(
            jax.ShapeDtypeStruct((N, DN), jnp.float32),
            jax.ShapeDtypeStruct((N, DE), jnp.float32),
        ),
        mesh=mesh,
        scratch_types=[
            pltpu.VMEM((PER_W, DEG), jnp.int32),
            pltpu.VMEM((PER_W, DEG), jnp.int32),
            pltpu.VMEM((PER_W * DEG,), jnp.int32),
            pltpu.VMEM((PER_W * DEG,), jnp.int32),
            pltpu.VMEM((2, ROWS, DN), jnp.float32),
            pltpu.VMEM((2, ROWS, DE), jnp.float32),
            pltpu.VMEM((2, ROWS), jnp.int32),
            pltpu.VMEM((A, DN), jnp.float32),
            pltpu.VMEM((A, DE), jnp.float32),
            pltpu.VMEM_SHARED((NS * 2 * A, DN), jnp.float32),
            pltpu.VMEM_SHARED((NS * 2 * A, DE), jnp.float32),
            pltpu.SemaphoreType.DMA((2,)),
            pltpu.SemaphoreType.DMA((2,)),
            pltpu.SemaphoreType.DMA((2,)),
            pltpu.SemaphoreType.DMA((2,)),
            pltpu.SemaphoreType.DMA((2,)),
        ],
        compiler_params=pltpu.CompilerParams(use_tc_tiling_on_sc=False),
    )


def _dot_t(x, w):
    return lax.dot_general(x, w, (((1,), (1,)), ((), ())),
                           preferred_element_type=jnp.float32)


def _onehot_t(mol_row):
    seg = lax.broadcasted_iota(jnp.int32, (NMOL, BLK), 0)
    return jnp.where(mol_row == seg, 1.0, 0.0)


def _tc1_body(ar_ref, asum_ref, bsum_ref, mol_ref, wdeg_ref, wself_ref,
              bias_ref, wout0_ref, bout0_ref, act_ref, stats_ref, fp0_ref):
    b = pl.program_id(0)
    ar = ar_ref[...]
    wdeg = wdeg_ref[...]
    wa = wdeg[:, :DN]
    wb = wdeg[:, DN:]
    wc = wa + wself_ref[...]
    act = (_dot_t(asum_ref[...], wa) + _dot_t(bsum_ref[...], wb)
           + _dot_t(ar, wc) + bias_ref[...])
    act_ref[...] = act

    psum = jnp.sum(act, axis=0, keepdims=True)
    psq = jnp.sum(act * act, axis=0, keepdims=True)

    logits = _dot_t(ar, wout0_ref[...]) + bout0_ref[...]
    m = jnp.max(logits, axis=1, keepdims=True)
    ex = jnp.exp(logits - m)
    soft = ex / jnp.sum(ex, axis=1, keepdims=True)
    oht = _onehot_t(mol_ref[0])
    fp_part = lax.dot_general(oht, soft, (((1,), (0,)), ((), ())),
                              preferred_element_type=jnp.float32)

    @pl.when(b == 0)
    def _():
        stats_ref[...] = jnp.zeros((2, DN), jnp.float32)
        fp0_ref[...] = jnp.zeros((NMOL, DOUT), jnp.float32)

    stats_ref[0:1, :] += psum
    stats_ref[1:2, :] += psq
    fp0_ref[...] += fp_part


_tc1 = pl.pallas_call(
    _tc1_body,
    grid=(NB,),
    in_specs=[
        pl.BlockSpec((BLK, DN), lambda b: (b, 0)),
        pl.BlockSpec((BLK, DN), lambda b: (b, 0)),
        pl.BlockSpec((BLK, DE), lambda b: (b, 0)),
        pl.BlockSpec((1, 1, BLK), lambda b: (b, 0, 0)),
        pl.BlockSpec((DOUT, DN + DE), lambda b: (0, 0)),
        pl.BlockSpec((DOUT, DN), lambda b: (0, 0)),
        pl.BlockSpec((1, DOUT), lambda b: (0, 0)),
        pl.BlockSpec((DOUT, DN), lambda b: (0, 0)),
        pl.BlockSpec((1, DOUT), lambda b: (0, 0)),
    ],
    out_specs=[
        pl.BlockSpec((BLK, DN), lambda b: (b, 0)),
        pl.BlockSpec((2, DN), lambda b: (0, 0)),
        pl.BlockSpec((NMOL, DOUT), lambda b: (0, 0)),
    ],
    out_shape=[
        jax.ShapeDtypeStruct((N, DN), jnp.float32),
        jax.ShapeDtypeStruct((2, DN), jnp.float32),
        jax.ShapeDtypeStruct((NMOL, DOUT), jnp.float32),
    ],
)


def _tc2_body(act_ref, mol_ref, stats_ref, fp0_ref, wout1_ref, bout1_ref,
              out_ref):
    b = pl.program_id(0)
    mean = stats_ref[0:1, :] * (1.0 / N)
    var = stats_ref[1:2, :] * (1.0 / N) - mean * mean
    h = jnp.maximum((act_ref[...] - mean) * lax.rsqrt(var + 1e-5), 0.0)
    logits = _dot_t(h, wout1_ref[...]) + bout1_ref[...]
    m = jnp.max(logits, axis=1, keepdims=True)
    ex = jnp.exp(logits - m)
    soft = ex / jnp.sum(ex, axis=1, keepdims=True)
    oht = _onehot_t(mol_ref[0])
    fp_part = lax.dot_general(oht, soft, (((1,), (0,)), ((), ())),
                              preferred_element_type=jnp.float32)

    @pl.when(b == 0)
    def _():
        out_ref[...] = fp0_ref[...]

    out_ref[...] += fp_part


_tc2 = pl.pallas_call(
    _tc2_body,
    grid=(NB,),
    in_specs=[
        pl.BlockSpec((BLK, DN), lambda b: (b, 0)),
        pl.BlockSpec((1, 1, BLK), lambda b: (b, 0, 0)),
        pl.BlockSpec((2, DN), lambda b: (0, 0)),
        pl.BlockSpec((NMOL, DOUT), lambda b: (0, 0)),
        pl.BlockSpec((DOUT, DOUT), lambda b: (0, 0)),
        pl.BlockSpec((1, DOUT), lambda b: (0, 0)),
    ],
    out_specs=pl.BlockSpec((NMOL, DOUT), lambda b: (0, 0)),
    out_shape=jax.ShapeDtypeStruct((NMOL, DOUT), jnp.float32),
)


def kernel(atom_repr, bond_repr, atom_nbr_idx, bond_nbr_idx, mol_ids,
           W_deg, W_self, bias, W_out0, b_out0, W_out1, b_out1):
    asum, bsum = _get_sc_kernel()(atom_repr, bond_repr,
                                  atom_nbr_idx, bond_nbr_idx)

    mol3 = mol_ids.astype(jnp.int32).reshape(NB, 1, BLK)
    act, stats, fp0 = _tc1(atom_repr, asum, bsum, mol3, W_deg, W_self, bias,
                           W_out0, b_out0.reshape(1, DOUT))
    return _tc2(act, mol3, stats, fp0, W_out1, b_out1.reshape(1, DOUT))
